# baseline (speedup 1.0000x reference)
import functools

import jax
import jax.numpy as jnp
from jax import lax
from jax.experimental import pallas as pl
from jax.experimental.pallas import tpu as pltpu
from jax.experimental.pallas import tpu_sc as plsc

B = 8
D = 8
H = 288
W = 512
HW = H * W
TPB = 4
P = HW // TPB
CH = 4096
NCH = P // CH
UNROLL = 2
NI = CH // (16 * UNROLL)
UNROLL2 = 2
NI2 = CH // (16 * UNROLL2)
NCLS = 5

DELTA_V = 1.0
DELTA_D = 6.0


def _newton_sqrt(x):
    xi = plsc.bitcast(x, jnp.int32)
    yi = jnp.int32(0x5F3759DF) - lax.shift_right_logical(xi, 1)
    y = plsc.bitcast(yi, jnp.float32)
    for _ in range(2):
        y = y * (1.5 - 0.5 * x * y * y)
    return x * y


@functools.partial(
    pl.kernel,
    out_type=jax.ShapeDtypeStruct((B, 96), jnp.float32),
    mesh=plsc.VectorSubcoreMesh(core_axis_name="c", subcore_axis_name="s"),
    compiler_params=pltpu.CompilerParams(needs_layout_passes=False),
    scratch_types=[
        pltpu.VMEM((2, D, 8, W), jnp.float32),
        pltpu.VMEM((P // W, W), jnp.int32),
        pltpu.VMEM((64,), jnp.float32),
        pltpu.VMEM((16,), jnp.float32),
        pltpu.VMEM((128,), jnp.float32),
        pltpu.VMEM((16,), jnp.float32),
        pltpu.VMEM((64,), jnp.float32),
        pltpu.VMEM((64,), jnp.float32),
        pltpu.VMEM((96,), jnp.float32),
        pltpu.VMEM_SHARED((16, 64), jnp.float32),
        pltpu.SemaphoreType.DMA,
        pltpu.SemaphoreType.DMA,
        pltpu.SemaphoreType.DMA,
    ],
)
def _sc_stats(tgt_hbm, emb_hbm, out_hbm, ebuf, lall, stab, ctab, meantab,
              safetab, tmp64, comb, outbuf, shared, semA, semB, semL):
    sems = (semA, semB)
    c = lax.axis_index("c")
    s = lax.axis_index("s")
    bq = s // TPB
    b = c * 4 + bq
    q = s - bq * TPB
    ROWS = P // W
    base_r = q * ROWS

    ldesc = pltpu.async_copy(tgt_hbm.at[b, pl.ds(base_r, ROWS), :], lall, semL)

    def issue(ch, slot):
        hr = base_r + ch * 8
        return pltpu.async_copy(
            emb_hbm.at[b, :, pl.ds(hr, 8), :], ebuf.at[slot], sems[slot])

    pending = issue(0, 0)

    zeros = jnp.zeros((16,), jnp.float32)
    for k in range(4):
        stab[pl.ds(16 * k, 16)] = zeros
    ctab[...] = zeros
    ldesc.wait()

    CLS_SPLIT = ((1, 2, 3), (4, 5))
    sums = {cl: [zeros for _ in range(D)] for cl in range(1, NCLS + 1)}
    cnts = {cl: zeros for cl in range(1, NCLS + 1)}
    for ch in range(NCH):
        slot = ch & 1
        crow = ch * 8

        pending.wait()
        pending = issue(ch + 1, (ch + 1) & 1) if ch < NCH - 1 else issue(0, 1)

        for cls_group in CLS_SPLIT:
            def grp1(i, carry, slot=slot, crow=crow, cls_group=cls_group):
                gsums, gcnts = carry
                gsums = [list(row) for row in gsums]
                gcnts = list(gcnts)
                for u in range(UNROLL):
                    off = i * (16 * UNROLL) + u * 16
                    r = lax.shift_right_logical(off, 9)
                    w = off & 511
                    lab = lall[crow + r, pl.ds(w, 16)]
                    es = [ebuf[slot, d, r, pl.ds(w, 16)] for d in range(D)]
                    for gi, cl in enumerate(cls_group):
                        mf = jnp.where(lab == cl, 1.0, 0.0)
                        gcnts[gi] = gcnts[gi] + mf
                        for d in range(D):
                            gsums[gi][d] = gsums[gi][d] + mf * es[d]
                return tuple(tuple(row) for row in gsums), tuple(gcnts)

            carry0 = (tuple(tuple(sums[cl]) for cl in cls_group),
                      tuple(cnts[cl] for cl in cls_group))
            gsums, gcnts = lax.fori_loop(0, NI, grp1, carry0)
            for gi, cl in enumerate(cls_group):
                sums[cl] = list(gsums[gi])
                cnts[cl] = gcnts[gi]

    for cl in range(1, NCLS + 1):
        for d in range(D):
            plsc.addupdate_scatter(stab, [jnp.full((16,), cl * 8 + d, jnp.int32)],
                                   sums[cl][d])
        plsc.addupdate_scatter(stab, [jnp.full((16,), 48 + cl, jnp.int32)],
                               cnts[cl])

    pltpu.sync_copy(stab, shared.at[s])
    plsc.subcore_barrier()
    r0 = bq * TPB
    pltpu.sync_copy(shared.at[r0], comb)
    for j in range(1, TPB):
        pltpu.sync_copy(shared.at[r0 + j], tmp64)
        for k in range(4):
            comb[pl.ds(16 * k, 16)] = comb[pl.ds(16 * k, 16)] + tmp64[pl.ds(16 * k, 16)]

    cntv = comb[pl.ds(48, 16)]
    safetab[...] = jnp.maximum(cntv, 1.0)
    iota16 = lax.iota(jnp.int32, 16)
    cls_clamped = jnp.minimum(iota16, 5)
    safe_c = plsc.load_gather(safetab, [cls_clamped])
    for d in range(D):
        sums_d = plsc.load_gather(comb, [cls_clamped * 8 + d])
        meantab[pl.ds(16 * d, 16)] = sums_d / safe_c

    caccs = [zeros for _ in range(NCLS)]
    for ch in range(NCH):
        slot = (ch + 1) & 1
        crow = ch * 8

        def grp2(i, carry, slot=slot, crow=crow):
            caccs = list(carry)
            for u in range(UNROLL2):
                off = i * (16 * UNROLL2) + u * 16
                r = lax.shift_right_logical(off, 9)
                w = off & 511
                lab = lall[crow + r, pl.ds(w, 16)]
                nsq = jnp.full((16,), 1e-12, jnp.float32)
                for d in range(D):
                    e = ebuf[slot, d, r, pl.ds(w, 16)]
                    m = plsc.load_gather(meantab, [lab + 16 * d])
                    df = e - m
                    nsq = nsq + df * df
                nrm = _newton_sqrt(nsq)
                t = jnp.maximum(nrm - DELTA_V, 0.0)
                cc = t * t
                for li in range(NCLS):
                    caccs[li] = caccs[li] + jnp.where(lab == (li + 1), cc, 0.0)
            return tuple(caccs)

        pending.wait()
        if ch < NCH - 1:
            pending = issue(ch + 1, ch & 1)
        caccs = lax.fori_loop(0, NI2, grp2, tuple(caccs))

    for li in range(NCLS):
        plsc.addupdate_scatter(ctab, [jnp.full((16,), li + 1, jnp.int32)],
                               caccs[li])

    plsc.subcore_barrier()
    pltpu.sync_copy(ctab, shared.at[s, pl.ds(0, 16)])
    plsc.subcore_barrier()

    @pl.when(q == 0)
    def _():
        csum = jnp.zeros((16,), jnp.float32)
        for j in range(TPB):
            pltpu.sync_copy(shared.at[r0 + j, pl.ds(0, 16)], tmp64.at[pl.ds(0, 16)])
            csum = csum + tmp64[pl.ds(0, 16)]
        outbuf[pl.ds(0, 16)] = comb[pl.ds(48, 16)]
        outbuf[pl.ds(16, 16)] = csum
        for g in range(3):
            cls_g = (iota16 >> 3) + 2 * g
            dim_g = iota16 & 7
            vals = plsc.load_gather(meantab, [dim_g * 16 + cls_g])
            outbuf[pl.ds(32 + 16 * g, 16)] = vals
        outbuf[pl.ds(80, 16)] = jnp.zeros((16,), jnp.float32)
        pltpu.sync_copy(outbuf, out_hbm.at[b])


def _combine_body(stats_ref, out_ref):
    st = stats_ref[...]
    cnt = st[:, 1:6]
    contrib = st[:, 17:22]
    means = st[:, 40:80].reshape(B, 5, D)
    valid = (cnt > 1.0).astype(jnp.float32)
    dist_sum = jnp.sum(valid * contrib)
    point_count = jnp.sum(valid * cnt)
    g = lax.dot_general(means, means, (((2,), (2,)), ((0,), (0,))))
    n2 = jnp.sum(means * means, axis=-1)
    d2 = jnp.maximum(n2[:, :, None] + n2[:, None, :] - 2.0 * g, 0.0)
    dmat = jnp.sqrt(d2 + 1e-12)
    pen = jnp.maximum(DELTA_D - dmat, 0.0) ** 2
    tri = jnp.triu(jnp.ones((5, 5), jnp.float32), k=1)
    w = valid[:, :, None] * valid[:, None, :] * tri[None]
    den = jnp.sum(w, axis=(1, 2))
    bval = (jnp.sum(valid, axis=1) > 1.0).astype(jnp.float32)
    var_b = jnp.sum(pen * w, axis=(1, 2)) / jnp.maximum(den, 1.0)
    var_vals = jnp.sum(bval * var_b)
    var_cnt = jnp.sum(bval)
    dist_loss = jnp.where(point_count > 0, dist_sum / jnp.maximum(point_count, 1.0), 0.0)
    var_loss = jnp.where(var_cnt > 0, var_vals / jnp.maximum(var_cnt, 1.0), 0.0)
    out_ref[...] = jnp.reshape(dist_loss + var_loss, (1, 1))


_combine = pl.pallas_call(
    _combine_body,
    out_shape=jax.ShapeDtypeStruct((1, 1), jnp.float32),
)


def kernel(targets, embedding_vector):
    stats = _sc_stats(targets.astype(jnp.int32), embedding_vector)
    return _combine(stats)[0, 0]


# speedup vs baseline: 1.0080x; 1.0080x over previous
import functools

import jax
import jax.numpy as jnp
from jax import lax
from jax.experimental import pallas as pl
from jax.experimental.pallas import tpu as pltpu
from jax.experimental.pallas import tpu_sc as plsc

B = 8
D = 8
H = 288
W = 512
HW = H * W
TPB = 4
P = HW // TPB
CH = 4096
NCH = P // CH
UNROLL = 2
NI = CH // (16 * UNROLL)
UNROLL2 = 2
NI2 = CH // (16 * UNROLL2)
NCLS = 5

DELTA_V = 1.0
DELTA_D = 6.0


def _newton_sqrt(x):
    xi = plsc.bitcast(x, jnp.int32)
    yi = jnp.int32(0x5F3759DF) - lax.shift_right_logical(xi, 1)
    y = plsc.bitcast(yi, jnp.float32)
    for _ in range(2):
        y = y * (1.5 - 0.5 * x * y * y)
    return x * y


@functools.partial(
    pl.kernel,
    out_type=jax.ShapeDtypeStruct((B, 96), jnp.float32),
    mesh=plsc.VectorSubcoreMesh(core_axis_name="c", subcore_axis_name="s"),
    compiler_params=pltpu.CompilerParams(needs_layout_passes=False),
    scratch_types=[
        pltpu.VMEM((3, D, 8, W), jnp.float32),
        pltpu.VMEM((3, 8, W), jnp.int32),
        pltpu.VMEM((64,), jnp.float32),
        pltpu.VMEM((16,), jnp.float32),
        pltpu.VMEM((128,), jnp.float32),
        pltpu.VMEM((16,), jnp.float32),
        pltpu.VMEM((64,), jnp.float32),
        pltpu.VMEM((64,), jnp.float32),
        pltpu.VMEM((96,), jnp.float32),
        pltpu.VMEM_SHARED((16, 64), jnp.float32),
        pltpu.SemaphoreType.DMA,
        pltpu.SemaphoreType.DMA,
        pltpu.SemaphoreType.DMA,
    ],
)
def _sc_stats(tgt_hbm, emb_hbm, out_hbm, ebuf, lbuf, stab, ctab, meantab,
              safetab, tmp64, comb, outbuf, shared, semA, semB, semC):
    sems = (semA, semB, semC)
    c = lax.axis_index("c")
    s = lax.axis_index("s")
    bq = s // TPB
    b = c * 4 + bq
    q = s - bq * TPB
    ROWS = P // W
    base_r = q * ROWS

    def issue(ch, slot):
        hr = base_r + ch * 8
        return (
            pltpu.async_copy(
                emb_hbm.at[b, :, pl.ds(hr, 8), :], ebuf.at[slot], sems[slot]),
            pltpu.async_copy(
                tgt_hbm.at[b, pl.ds(hr, 8), :], lbuf.at[slot], sems[slot]),
        )

    inflight = {0: issue(0, 0), 1: issue(1, 1)}

    zeros = jnp.zeros((16,), jnp.float32)
    for k in range(4):
        stab[pl.ds(16 * k, 16)] = zeros
    ctab[...] = zeros

    CLS_SPLIT = ((1, 2, 3), (4, 5))
    sums = {cl: [zeros for _ in range(D)] for cl in range(1, NCLS + 1)}
    cnts = {cl: zeros for cl in range(1, NCLS + 1)}
    for ch in range(NCH):
        slot = ch % 3

        for dsc in inflight.pop(ch):
            dsc.wait()
        nxt = ch + 2
        if nxt < NCH:
            inflight[nxt] = issue(nxt, nxt % 3)
        else:
            inflight[nxt] = issue(nxt - NCH, nxt % 3)

        for cls_group in CLS_SPLIT:
            def grp1(i, carry, slot=slot, cls_group=cls_group):
                gsums, gcnts = carry
                gsums = [list(row) for row in gsums]
                gcnts = list(gcnts)
                for u in range(UNROLL):
                    off = i * (16 * UNROLL) + u * 16
                    r = lax.shift_right_logical(off, 9)
                    w = off & 511
                    lab = lbuf[slot, r, pl.ds(w, 16)]
                    es = [ebuf[slot, d, r, pl.ds(w, 16)] for d in range(D)]
                    for gi, cl in enumerate(cls_group):
                        mf = jnp.where(lab == cl, 1.0, 0.0)
                        gcnts[gi] = gcnts[gi] + mf
                        for d in range(D):
                            gsums[gi][d] = gsums[gi][d] + mf * es[d]
                return tuple(tuple(row) for row in gsums), tuple(gcnts)

            carry0 = (tuple(tuple(sums[cl]) for cl in cls_group),
                      tuple(cnts[cl] for cl in cls_group))
            gsums, gcnts = lax.fori_loop(0, NI, grp1, carry0)
            for gi, cl in enumerate(cls_group):
                sums[cl] = list(gsums[gi])
                cnts[cl] = gcnts[gi]

    for cl in range(1, NCLS + 1):
        for d in range(D):
            plsc.addupdate_scatter(stab, [jnp.full((16,), cl * 8 + d, jnp.int32)],
                                   sums[cl][d])
        plsc.addupdate_scatter(stab, [jnp.full((16,), 48 + cl, jnp.int32)],
                               cnts[cl])

    pltpu.sync_copy(stab, shared.at[s])
    plsc.subcore_barrier()
    r0 = bq * TPB
    pltpu.sync_copy(shared.at[r0], comb)
    for j in range(1, TPB):
        pltpu.sync_copy(shared.at[r0 + j], tmp64)
        for k in range(4):
            comb[pl.ds(16 * k, 16)] = comb[pl.ds(16 * k, 16)] + tmp64[pl.ds(16 * k, 16)]

    cntv = comb[pl.ds(48, 16)]
    safetab[...] = jnp.maximum(cntv, 1.0)
    iota16 = lax.iota(jnp.int32, 16)
    cls_clamped = jnp.minimum(iota16, 5)
    safe_c = plsc.load_gather(safetab, [cls_clamped])
    for d in range(D):
        sums_d = plsc.load_gather(comb, [cls_clamped * 8 + d])
        meantab[pl.ds(16 * d, 16)] = sums_d / safe_c

    caccs = [zeros for _ in range(NCLS)]
    for ch in range(NCH):
        slot = ch % 3

        def grp2(i, carry, slot=slot):
            caccs = list(carry)
            for u in range(UNROLL2):
                off = i * (16 * UNROLL2) + u * 16
                r = lax.shift_right_logical(off, 9)
                w = off & 511
                lab = lbuf[slot, r, pl.ds(w, 16)]
                nsq = jnp.full((16,), 1e-12, jnp.float32)
                for d in range(D):
                    e = ebuf[slot, d, r, pl.ds(w, 16)]
                    m = plsc.load_gather(meantab, [lab + 16 * d])
                    df = e - m
                    nsq = nsq + df * df
                nrm = _newton_sqrt(nsq)
                t = jnp.maximum(nrm - DELTA_V, 0.0)
                cc = t * t
                for li in range(NCLS):
                    caccs[li] = caccs[li] + jnp.where(lab == (li + 1), cc, 0.0)
            return tuple(caccs)

        for dsc in inflight.pop(NCH + ch):
            dsc.wait()
        if ch + 2 < NCH:
            inflight[NCH + ch + 2] = issue(ch + 2, (ch + 2) % 3)
        caccs = lax.fori_loop(0, NI2, grp2, tuple(caccs))

    for li in range(NCLS):
        plsc.addupdate_scatter(ctab, [jnp.full((16,), li + 1, jnp.int32)],
                               caccs[li])

    plsc.subcore_barrier()
    pltpu.sync_copy(ctab, shared.at[s, pl.ds(0, 16)])
    plsc.subcore_barrier()

    @pl.when(q == 0)
    def _():
        csum = jnp.zeros((16,), jnp.float32)
        for j in range(TPB):
            pltpu.sync_copy(shared.at[r0 + j, pl.ds(0, 16)], tmp64.at[pl.ds(0, 16)])
            csum = csum + tmp64[pl.ds(0, 16)]
        outbuf[pl.ds(0, 16)] = comb[pl.ds(48, 16)]
        outbuf[pl.ds(16, 16)] = csum
        for g in range(3):
            cls_g = (iota16 >> 3) + 2 * g
            dim_g = iota16 & 7
            vals = plsc.load_gather(meantab, [dim_g * 16 + cls_g])
            outbuf[pl.ds(32 + 16 * g, 16)] = vals
        outbuf[pl.ds(80, 16)] = jnp.zeros((16,), jnp.float32)
        pltpu.sync_copy(outbuf, out_hbm.at[b])


def _combine_body(stats_ref, out_ref):
    st = stats_ref[...]
    cnt = st[:, 1:6]
    contrib = st[:, 17:22]
    means = st[:, 40:80].reshape(B, 5, D)
    valid = (cnt > 1.0).astype(jnp.float32)
    dist_sum = jnp.sum(valid * contrib)
    point_count = jnp.sum(valid * cnt)
    g = lax.dot_general(means, means, (((2,), (2,)), ((0,), (0,))))
    n2 = jnp.sum(means * means, axis=-1)
    d2 = jnp.maximum(n2[:, :, None] + n2[:, None, :] - 2.0 * g, 0.0)
    dmat = jnp.sqrt(d2 + 1e-12)
    pen = jnp.maximum(DELTA_D - dmat, 0.0) ** 2
    tri = jnp.triu(jnp.ones((5, 5), jnp.float32), k=1)
    w = valid[:, :, None] * valid[:, None, :] * tri[None]
    den = jnp.sum(w, axis=(1, 2))
    bval = (jnp.sum(valid, axis=1) > 1.0).astype(jnp.float32)
    var_b = jnp.sum(pen * w, axis=(1, 2)) / jnp.maximum(den, 1.0)
    var_vals = jnp.sum(bval * var_b)
    var_cnt = jnp.sum(bval)
    dist_loss = jnp.where(point_count > 0, dist_sum / jnp.maximum(point_count, 1.0), 0.0)
    var_loss = jnp.where(var_cnt > 0, var_vals / jnp.maximum(var_cnt, 1.0), 0.0)
    out_ref[...] = jnp.reshape(dist_loss + var_loss, (1, 1))


_combine = pl.pallas_call(
    _combine_body,
    out_shape=jax.ShapeDtypeStruct((1, 1), jnp.float32),
)


def kernel(targets, embedding_vector):
    stats = _sc_stats(targets.astype(jnp.int32), embedding_vector)
    return _combine(stats)[0, 0]
